# baseline (speedup 1.0000x reference)
import jax
import jax.numpy as jnp
from jax import lax
from jax.experimental import pallas as pl
from jax.experimental.pallas import tpu as pltpu
from jax.experimental.pallas import tpu_sc as plsc

GRID = 32
RSTEP = 0.1
RMAX = 3.0
NATOMS = 128
NRAD = 64
L = 16


def _splat(vec, j):
  return jnp.full((L,), vec[j], dtype=vec.dtype)


def _sc_body(pack_h, dens_h, out_h, pack_v, dens_v, plane_v, sem):
  cid = lax.axis_index("c")
  sid = lax.axis_index("s")
  wid = sid * 2 + cid

  c1 = pltpu.async_copy(dens_h, dens_v, sem)
  c2 = pltpu.async_copy(pack_h, pack_v, sem)
  c1.wait()
  c2.wait()

  gv = pack_v[pl.ds(3 * NATOMS, L)]
  g00 = _splat(gv, 0)
  g01 = _splat(gv, 1)
  g02 = _splat(gv, 2)
  g11 = _splat(gv, 3)
  g12 = _splat(gv, 4)
  g22 = _splat(gv, 5)
  ngd = _splat(gv, 6)
  invg11 = _splat(gv, 7)
  inv_rstep = _splat(gv, 8)

  zf = jnp.full((L,), wid, dtype=jnp.int32).astype(jnp.float32)
  iota = lax.iota(jnp.int32, L)
  xf0 = iota.astype(jnp.float32)
  xf1 = (iota + 16).astype(jnp.float32)
  g00x0 = g00 * xf0
  g00x1 = g00 * xf1

  zero16 = jnp.zeros((L,), jnp.float32)

  def zero_body(r, _):
    plane_v[r] = zero16
    return _

  lax.fori_loop(0, 2 * GRID + 4, zero_body, None)

  rmax2 = jnp.full((L,), RMAX * RMAX, jnp.float32)
  rmax2_pad = jnp.full((L,), RMAX * RMAX + 1e-3, jnp.float32)
  half = jnp.full((L,), 0.5, jnp.float32)
  three_half = jnp.full((L,), 1.5, jnp.float32)
  magic = jnp.full((L,), 0x5F3759DF, jnp.int32)
  one_i = jnp.full((L,), 1, jnp.int32)
  zero_i = jnp.full((L,), 0, jnp.int32)

  def newton_rsqrt(a):
    bits = plsc.bitcast(a, jnp.int32)
    y0 = plsc.bitcast(magic - lax.shift_right_logical(bits, 1), jnp.float32)
    hx = half * a
    y0 = y0 * (three_half - hx * y0 * y0)
    y0 = y0 * (three_half - hx * y0 * y0)
    return y0

  def chunk_body(c, _):
    base = c * L
    axv = pack_v[pl.ds(base, L)]
    ayv = pack_v[pl.ds(NATOMS + base, L)]
    azv = pack_v[pl.ds(2 * NATOMS + base, L)]

    dzv = zf - azv
    cdzv = g22 * dzv
    remy = rmax2_pad - cdzv * cdzv
    ok = remy >= 0.0
    remy_nn = jnp.maximum(remy, 0.0)
    sy = remy_nn * newton_rsqrt(remy_nn)
    sy = sy * 1.00002 + 1e-3
    sy = jnp.where(ok, sy, -1.0)
    cyv = ngd * dzv
    hw = sy * invg11
    ylo_f = jnp.maximum(ayv + cyv - hw, 0.0)
    yhi_f = jnp.minimum(ayv + cyv + hw, GRID - 1.0)
    ilo = ylo_f.astype(jnp.int32)
    ylov = ilo + jnp.where(ilo.astype(jnp.float32) < ylo_f, one_i, zero_i)
    ycntv = yhi_f.astype(jnp.int32) - ylov + 1

    for j in range(L):
      ycnt_s = ycntv[j]

      @pl.when(ycnt_s > 0)
      def _():
        n = base + j
        nv = jnp.full((L,), n, jnp.int32)
        axs = _splat(axv, j)
        ays = _splat(ayv, j)
        dzs = _splat(dzv, j)
        cdzs = _splat(cdzv, j)
        ylo_s = ylov[j]
        cdz2 = cdzs * cdzs
        g12dz = g12 * dzs
        g02dz = g02 * dzs
        g00ax = g00 * axs

        @plsc.parallel_loop(0, ycnt_s, unroll=1)
        def row_body(yi):
          y = ylo_s + yi

          def one_row(yy):
            dyv = jnp.full((L,), yy, jnp.int32).astype(jnp.float32) - ays
            cdy = g12dz + g11 * dyv
            cyz2 = cdz2 + cdy * cdy
            rowbase = (g02dz + g01 * dyv) - g00ax
            r = yy * 2

            def do_half(hh, g00xf):
              cdx = rowbase + g00xf
              d2 = cdx * cdx + cyz2
              m = d2 <= rmax2
              y0 = newton_rsqrt(d2)
              dist = d2 * y0
              rad = dist * inv_rstep
              il_raw = rad.astype(jnp.int32)
              wh = rad - il_raw.astype(jnp.float32)
              il = jnp.minimum(il_raw, NRAD - 1)
              ih = jnp.minimum(il_raw + 1, NRAD - 1)
              dl = plsc.load_gather(dens_v, [nv, il])
              dh = plsc.load_gather(dens_v, [nv, ih])
              dens = dl + wh * (dh - dl)
              contrib = jnp.where(m, dens, 0.0)
              plsc.addupdate(plane_v.at[r + hh], contrib)

            do_half(0, g00x0)
            do_half(1, g00x1)

          one_row(y)

    return _

  lax.fori_loop(0, NATOMS // L, chunk_body, None)

  pltpu.async_copy(plane_v.at[pl.ds(0, 2 * GRID)],
                   out_h.at[wid], sem).wait()


def kernel(coordinates, active, occupancies, lmax, radial_densities,
           grid_to_cartesian):
  del lmax
  dtype = jnp.float32
  coords = coordinates[0].astype(dtype)
  ax = coords[:, 0]
  ay = coords[:, 1]
  az = coords[:, 2]
  occ = (occupancies[0] * active[0].astype(dtype)).astype(dtype)
  dens = radial_densities[0].astype(dtype) * occ[:, None]

  g = grid_to_cartesian.astype(dtype)
  rstep = jnp.asarray(RSTEP, dtype)
  gv = jnp.stack([
      g[0, 0], g[0, 1], g[0, 2], g[1, 1], g[1, 2], g[2, 2],
      -g[1, 2] / g[1, 1], 1.0 / jnp.abs(g[1, 1]), 1.0 / rstep,
      jnp.zeros((), dtype), jnp.zeros((), dtype), jnp.zeros((), dtype),
      jnp.zeros((), dtype), jnp.zeros((), dtype), jnp.zeros((), dtype),
      jnp.zeros((), dtype),
  ])
  pack = jnp.concatenate([ax, ay, az, gv])

  mesh = plsc.VectorSubcoreMesh(core_axis_name="c", subcore_axis_name="s")
  run = pl.kernel(
      _sc_body,
      out_type=jax.ShapeDtypeStruct((GRID, 2 * GRID, L), dtype),
      mesh=mesh,
      compiler_params=pltpu.CompilerParams(needs_layout_passes=False),
      scratch_types=[
          pltpu.VMEM((3 * NATOMS + L,), dtype),
          pltpu.VMEM((NATOMS, NRAD), dtype),
          pltpu.VMEM((2 * GRID + 4, L), dtype),
          pltpu.SemaphoreType.DMA,
      ],
  )
  out = run(pack, dens)
  return out.reshape((1, GRID, GRID, GRID))
